# baseline (speedup 1.0000x reference)
import functools

import jax
import jax.numpy as jnp
from jax import lax
from jax.experimental import pallas as pl
from jax.experimental.pallas import tpu as pltpu
from jax.experimental.pallas import tpu_sc as plsc

_SHAPE = (4096, 4096)
_FLAT = _SHAPE[0] * _SHAPE[1]
_K = 1048576
_NC, _NS = 2, 16
_NW = _NC * _NS
_BLK = 32768
_CH = _BLK * _NS
_NCHUNK = _FLAT // _NC // _CH
_NBLK = _FLAT // _BLK
_NR = 48
_PROWS = _K // 128

_mesh = plsc.VectorSubcoreMesh(core_axis_name="c", subcore_axis_name="s")


@functools.partial(
    pl.kernel,
    out_type=jax.ShapeDtypeStruct(_SHAPE, jnp.float32),
    mesh=_mesh,
    compiler_params=pltpu.CompilerParams(needs_layout_passes=False),
    scratch_types=[
        pltpu.MemorySpace.VMEM_SHARED((_CH + 16,), jnp.float32),
        pltpu.MemorySpace.VMEM_SHARED((_CH + 16,), jnp.float32),
        pltpu.VMEM((_NR, 128), jnp.int32),
        pltpu.VMEM((_NR, 128), jnp.int32),
        pltpu.VMEM((_NR, 128), jnp.float32),
        pltpu.VMEM((_NR, 128), jnp.float32),
        pltpu.VMEM((_NR, 128), jnp.int32),
        pltpu.VMEM((_NR, 128), jnp.int32),
        pltpu.VMEM((32,), jnp.int32),
        pltpu.SemaphoreType.DMA,
        pltpu.SemaphoreType.DMA,
        pltpu.SemaphoreType.DMA,
        pltpu.SemaphoreType.DMA,
        pltpu.SemaphoreType.DMA,
        pltpu.SemaphoreType.DMA,
        pltpu.SemaphoreType.DMA,
    ],
)
def _sc_scatter_add(tensor_hbm, val2d_hbm, idx2d_hbm, table_hbm, out_hbm,
                    sp_a, sp_b, idx_a, idx_b, val_a, val_b, lidx_a, lidx_b,
                    bnd_v, dld_a, dld_b, dst_a, dst_b, pld_a, pld_b, ssem):
    c = lax.axis_index("c")
    s = lax.axis_index("s")
    row_id = s * _NC + c
    sl_lo = s * _BLK

    pltpu.sync_copy(table_hbm.at[pl.ds(row_id * 32, 32)], bnd_v)
    bv0 = bnd_v[pl.ds(0, 16)]
    bv1 = bnd_v[pl.ds(16, 16)]

    trash = lax.iota(jnp.int32, 16) + _CH
    blk_u = jnp.uint32(_BLK)

    sps = (sp_a, sp_b)
    idxs = (idx_a, idx_b)
    vals = (val_a, val_b)
    lidxs = (lidx_a, lidx_b)
    dld = (dld_a, dld_b)
    dst = (dst_a, dst_b)
    pld = (pld_a, pld_b)

    dld_desc = [None, None]
    dst_desc = [None, None]
    pld_desc = [None, None]

    def chunk_lo_of(k):
        return (c * _NCHUNK + k) * _CH

    def start_dense_load(k):
        cur = k % 2
        row0 = ((c * _NCHUNK + k) * _NS + s) * 8
        dld_desc[cur] = [
            pltpu.async_copy(
                tensor_hbm.at[row0 + r],
                sps[cur].at[pl.ds(sl_lo + r * 4096, 4096)], dld[cur])
            for r in range(8)]

    def start_pair_load(k, r0):
        cur = k % 2
        d1 = pltpu.async_copy(idx2d_hbm.at[pl.ds(r0, _NR)], idxs[cur], pld[cur])
        d2 = pltpu.async_copy(val2d_hbm.at[pl.ds(r0, _NR)], vals[cur], pld[cur])
        pld_desc[cur] = (d1, d2)

    def compute_lidx(lo, hi, idxb, lidxb, blk_lo):
        def body(row, carry):
            for g in range(8):
                iv = idxb[row, pl.ds(g * 16, 16)]
                d = iv - blk_lo
                inb = plsc.bitcast(d, jnp.uint32) < blk_u
                lv = jnp.where(inb, d + sl_lo, trash)
                lidxb[row, pl.ds(g * 16, 16)] = lv
            return carry
        lax.fori_loop(lo, hi, body, 0)

    def issue_streams(lo, hi, valb, lidxb, spb):
        def body(row, carry):
            pltpu.async_copy(valb.at[row], spb.at[lidxb.at[row]], ssem,
                             add=True)
            return carry
        lax.fori_loop(lo, hi, body, 0)

    def drain_streams(lo, hi, valb, lidxb, spb):
        def body(row, carry):
            pltpu.make_async_copy(valb.at[row], spb.at[lidxb.at[row]],
                                  ssem).wait()
            return carry
        lax.fori_loop(lo, hi, body, 0)

    r0s = [pl.multiple_of(jnp.minimum((bv0[k] >> 10) << 3, _PROWS - _NR), 8)
           for k in range(_NCHUNK)]
    nrs = [((bv1[k] + 127) >> 7) - r0s[k] for k in range(_NCHUNK)]

    start_dense_load(0)
    start_pair_load(0, r0s[0])
    for d in pld_desc[0]:
        d.wait()
    compute_lidx(0, jnp.minimum(nrs[0], _NR), idxs[0], lidxs[0],
                 chunk_lo_of(0) + sl_lo)

    for k in range(_NCHUNK):
        cur = k % 2
        nxt = 1 - cur
        if k + 1 < _NCHUNK:
            if dst_desc[nxt] is not None:
                for d in dst_desc[nxt]:
                    d.wait()
                dst_desc[nxt] = None
            start_dense_load(k + 1)
            start_pair_load(k + 1, r0s[k + 1])

        blk_lo = chunk_lo_of(k) + sl_lo
        nr = nrs[k]
        nr1 = jnp.minimum(nr, _NR)

        for d in dld_desc[cur]:
            d.wait()
        issue_streams(0, nr1, vals[cur], lidxs[cur], sps[cur])

        if k + 1 < _NCHUNK:
            for d in pld_desc[nxt]:
                d.wait()
            pld_desc[nxt] = None
            compute_lidx(0, jnp.minimum(nrs[k + 1], _NR), idxs[nxt],
                         lidxs[nxt], chunk_lo_of(k + 1) + sl_lo)

        drain_streams(0, nr1, vals[cur], lidxs[cur], sps[cur])

        nbatch = (nr + _NR - 1) // _NR

        def rem_body(j, carry, cur=cur, r0=r0s[k], nr=nr, blk_lo=blk_lo):
            b0 = r0 + j * _NR
            rb = pl.multiple_of(jnp.minimum(b0, _PROWS - _NR), 8)
            ov = b0 - rb
            pltpu.sync_copy(idx2d_hbm.at[pl.ds(rb, _NR)], idxs[cur])
            pltpu.sync_copy(val2d_hbm.at[pl.ds(rb, _NR)], vals[cur])
            hi = ov + jnp.minimum(nr - j * _NR, _NR)
            compute_lidx(ov, hi, idxs[cur], lidxs[cur], blk_lo)
            issue_streams(ov, hi, vals[cur], lidxs[cur], sps[cur])
            drain_streams(ov, hi, vals[cur], lidxs[cur], sps[cur])
            return carry

        lax.fori_loop(1, nbatch, rem_body, 0)

        row0 = ((c * _NCHUNK + k) * _NS + s) * 8
        dst_desc[cur] = [
            pltpu.async_copy(
                sps[cur].at[pl.ds(sl_lo + r * 4096, 4096)],
                out_hbm.at[row0 + r], dst[cur])
            for r in range(8)]

    for ds_ in dst_desc:
        if ds_ is not None:
            for d in ds_:
                d.wait()


def kernel(tensor, values, indices):
    queries = jnp.arange(_NBLK + 1, dtype=jnp.int32) * _BLK
    bounds = jnp.searchsorted(indices, queries, side="left").astype(jnp.int32)
    w = jnp.arange(_NW)
    s_ = w // _NC
    c_ = w % _NC
    k_ = jnp.arange(_NCHUNK)
    ids = c_[:, None] * (_NBLK // _NC) + k_[None, :] * _NS + s_[:, None]
    table = jnp.concatenate([bounds[ids], bounds[ids + 1]],
                            axis=1).reshape(-1)

    idx2d = indices.reshape(_PROWS, 128)
    val2d = values.reshape(_PROWS, 128)

    return _sc_scatter_add(tensor, val2d, idx2d, table)


# speedup vs baseline: 1.0197x; 1.0197x over previous
import functools

import jax
import jax.numpy as jnp
from jax import lax
from jax.experimental import pallas as pl
from jax.experimental.pallas import tpu as pltpu
from jax.experimental.pallas import tpu_sc as plsc

_SHAPE = (4096, 4096)
_FLAT = _SHAPE[0] * _SHAPE[1]
_K = 1048576
_NC, _NS = 2, 16
_NW = _NC * _NS
_BLK = 32768
_NCHUNK = 16
_NBLK = _FLAT // _BLK
_NR = 48
_PROWS = _K // 128

_mesh = plsc.VectorSubcoreMesh(core_axis_name="c", subcore_axis_name="s")


@functools.partial(
    pl.kernel,
    out_type=jax.ShapeDtypeStruct(_SHAPE, jnp.float32),
    mesh=_mesh,
    compiler_params=pltpu.CompilerParams(needs_layout_passes=False),
    scratch_types=[
        pltpu.VMEM((8, 4096), jnp.float32),
        pltpu.VMEM((8, 4096), jnp.float32),
        pltpu.VMEM((_NR, 128), jnp.int32),
        pltpu.VMEM((_NR, 128), jnp.int32),
        pltpu.VMEM((_NR, 128), jnp.float32),
        pltpu.VMEM((_NR, 128), jnp.float32),
        pltpu.VMEM((32,), jnp.int32),
        pltpu.SemaphoreType.DMA,
        pltpu.SemaphoreType.DMA,
        pltpu.SemaphoreType.DMA,
        pltpu.SemaphoreType.DMA,
        pltpu.SemaphoreType.DMA,
        pltpu.SemaphoreType.DMA,
    ],
)
def _sc_scatter_add(tensor_hbm, val2d_hbm, idx2d_hbm, table_hbm, out_hbm,
                    den_a, den_b, idx_a, idx_b, val_a, val_b,
                    bnd_v, dld_a, dld_b, dst_a, dst_b, pld_a, pld_b):
    c = lax.axis_index("c")
    s = lax.axis_index("s")
    row_id = s * _NC + c

    pltpu.sync_copy(table_hbm.at[pl.ds(row_id * 32, 32)], bnd_v)
    bv0 = bnd_v[pl.ds(0, 16)]
    bv1 = bnd_v[pl.ds(16, 16)]

    blk_u = jnp.uint32(_BLK)
    t3 = tensor_hbm.reshape(_NBLK, 8, 4096)
    o3 = out_hbm.reshape(_NBLK, 8, 4096)

    dens = (den_a, den_b)
    idxs = (idx_a, idx_b)
    vals = (val_a, val_b)
    dld = (dld_a, dld_b)
    dstm = (dst_a, dst_b)
    pld = (pld_a, pld_b)

    dld_desc = [None, None]
    dst_desc = [None, None]
    pld_desc = [None, None]

    def blk_of(k):
        return (c * _NCHUNK + k) * _NS + s

    def start_dense_load(k):
        cur = k % 2
        dld_desc[cur] = pltpu.async_copy(t3.at[blk_of(k)], dens[cur],
                                         dld[cur])

    def start_pair_load(k, r0):
        cur = k % 2
        d1 = pltpu.async_copy(idx2d_hbm.at[pl.ds(r0, _NR)], idxs[cur],
                              pld[cur])
        d2 = pltpu.async_copy(val2d_hbm.at[pl.ds(r0, _NR)], vals[cur],
                              pld[cur])
        pld_desc[cur] = (d1, d2)

    def scatter_rows(lo, hi, idxb, valb, denb, blk_lo):
        def body(row, carry):
            for g in range(8):
                iv = idxb[row, pl.ds(g * 16, 16)]
                d = iv - blk_lo
                inb = plsc.bitcast(d, jnp.uint32) < blk_u
                r16 = lax.bitwise_and(lax.shift_right_logical(d, 12), 7)
                c16 = lax.bitwise_and(d, 4095)
                v16 = valb[row, pl.ds(g * 16, 16)]
                plsc.addupdate_scatter(denb, [r16, c16], v16, mask=inb)
            return carry
        lax.fori_loop(lo, hi, body, 0)

    r0s = [pl.multiple_of(jnp.minimum((bv0[k] >> 10) << 3, _PROWS - _NR), 8)
           for k in range(_NCHUNK)]
    nrs = [((bv1[k] + 127) >> 7) - r0s[k] for k in range(_NCHUNK)]

    start_dense_load(0)
    start_pair_load(0, r0s[0])

    for k in range(_NCHUNK):
        cur = k % 2
        nxt = 1 - cur
        if k + 1 < _NCHUNK:
            if dst_desc[nxt] is not None:
                dst_desc[nxt].wait()
                dst_desc[nxt] = None
            start_dense_load(k + 1)
            start_pair_load(k + 1, r0s[k + 1])

        blk_lo = blk_of(k) * _BLK
        nr = nrs[k]
        nr1 = jnp.minimum(nr, _NR)

        for d in pld_desc[cur]:
            d.wait()
        pld_desc[cur] = None
        dld_desc[cur].wait()

        scatter_rows(0, nr1, idxs[cur], vals[cur], dens[cur], blk_lo)

        nbatch = (nr + _NR - 1) // _NR

        def rem_body(j, carry, cur=cur, rb=r0s[k], nr=nr, blk_lo=blk_lo):
            b0 = rb + j * _NR
            rbj = pl.multiple_of(jnp.minimum(b0, _PROWS - _NR), 8)
            ov = b0 - rbj
            pltpu.sync_copy(idx2d_hbm.at[pl.ds(rbj, _NR)], idxs[cur])
            pltpu.sync_copy(val2d_hbm.at[pl.ds(rbj, _NR)], vals[cur])
            hi = ov + jnp.minimum(nr - j * _NR, _NR)
            scatter_rows(ov, hi, idxs[cur], vals[cur], dens[cur], blk_lo)
            return carry

        lax.fori_loop(1, nbatch, rem_body, 0)

        dst_desc[cur] = pltpu.async_copy(dens[cur], o3.at[blk_of(k)],
                                         dstm[cur])

    for d in dst_desc:
        if d is not None:
            d.wait()


def kernel(tensor, values, indices):
    queries = jnp.arange(_NBLK + 1, dtype=jnp.int32) * _BLK
    bounds = jnp.searchsorted(indices, queries, side="left").astype(jnp.int32)
    w = jnp.arange(_NW)
    s_ = w // _NC
    c_ = w % _NC
    k_ = jnp.arange(_NCHUNK)
    ids = c_[:, None] * (_NBLK // _NC) + k_[None, :] * _NS + s_[:, None]
    table = jnp.concatenate([bounds[ids], bounds[ids + 1]],
                            axis=1).reshape(-1)

    idx2d = indices.reshape(_PROWS, 128)
    val2d = values.reshape(_PROWS, 128)

    return _sc_scatter_add(tensor, val2d, idx2d, table)


# speedup vs baseline: 1.5804x; 1.5499x over previous
import functools

import jax
import jax.numpy as jnp
from jax import lax
from jax.experimental import pallas as pl
from jax.experimental.pallas import tpu as pltpu
from jax.experimental.pallas import tpu_sc as plsc

_SHAPE = (4096, 4096)
_FLAT = _SHAPE[0] * _SHAPE[1]
_K = 1048576
_NC, _NS = 2, 16
_NW = _NC * _NS
_BLK = 32768
_CH = _BLK * _NS
_NCHUNK = _FLAT // _NC // _CH
_NBLK = _FLAT // _BLK
_NR = 48
_PROWS = _K // 128

_mesh = plsc.VectorSubcoreMesh(core_axis_name="c", subcore_axis_name="s")


@functools.partial(
    pl.kernel,
    out_type=jax.ShapeDtypeStruct((_FLAT,), jnp.float32),
    mesh=_mesh,
    compiler_params=pltpu.CompilerParams(needs_layout_passes=False),
    scratch_types=[
        pltpu.MemorySpace.VMEM_SHARED((_CH + 16,), jnp.float32),
        pltpu.MemorySpace.VMEM_SHARED((_CH + 16,), jnp.float32),
        pltpu.VMEM((_NR, 128), jnp.int32),
        pltpu.VMEM((_NR, 128), jnp.int32),
        pltpu.VMEM((_NR, 128), jnp.float32),
        pltpu.VMEM((_NR, 128), jnp.float32),
        pltpu.VMEM((_NR, 128), jnp.int32),
        pltpu.VMEM((_NR, 128), jnp.int32),
        pltpu.VMEM((32,), jnp.int32),
        pltpu.SemaphoreType.DMA,
        pltpu.SemaphoreType.DMA,
        pltpu.SemaphoreType.DMA,
        pltpu.SemaphoreType.DMA,
        pltpu.SemaphoreType.DMA,
        pltpu.SemaphoreType.DMA,
        pltpu.SemaphoreType.DMA,
    ],
)
def _sc_scatter_add(tensor_hbm, val2d_hbm, idx2d_hbm, table_hbm, out_hbm,
                    sp_a, sp_b, idx_a, idx_b, val_a, val_b, lidx_a, lidx_b,
                    bnd_v, dld_a, dld_b, dst_a, dst_b, pld_a, pld_b, ssem):
    c = lax.axis_index("c")
    s = lax.axis_index("s")
    row_id = s * _NC + c
    sl_lo = s * _BLK

    pltpu.sync_copy(table_hbm.at[pl.ds(row_id * 32, 32)], bnd_v)
    bv0 = bnd_v[pl.ds(0, 16)]
    bv1 = bnd_v[pl.ds(16, 16)]

    trash = lax.iota(jnp.int32, 16) + _CH
    blk_u = jnp.uint32(_BLK)

    sps = (sp_a, sp_b)
    idxs = (idx_a, idx_b)
    vals = (val_a, val_b)
    lidxs = (lidx_a, lidx_b)
    dld = (dld_a, dld_b)
    dst = (dst_a, dst_b)
    pld = (pld_a, pld_b)

    dld_desc = [None, None]
    dst_desc = [None, None]
    pld_desc = [None, None]

    def chunk_lo_of(k):
        return (c * _NCHUNK + k) * _CH

    def start_dense_load(k):
        cur = k % 2
        hbm_lo = chunk_lo_of(k) + sl_lo
        dld_desc[cur] = pltpu.async_copy(
            tensor_hbm.at[pl.ds(hbm_lo, _BLK)],
            sps[cur].at[pl.ds(sl_lo, _BLK)], dld[cur])

    def start_pair_load(k, r0):
        cur = k % 2
        d1 = pltpu.async_copy(idx2d_hbm.at[pl.ds(r0, _NR)], idxs[cur], pld[cur])
        d2 = pltpu.async_copy(val2d_hbm.at[pl.ds(r0, _NR)], vals[cur], pld[cur])
        pld_desc[cur] = (d1, d2)

    def compute_lidx(lo, hi, idxb, lidxb, blk_lo):
        def body(row, carry):
            for g in range(8):
                iv = idxb[row, pl.ds(g * 16, 16)]
                d = iv - blk_lo
                inb = plsc.bitcast(d, jnp.uint32) < blk_u
                lv = jnp.where(inb, d + sl_lo, trash)
                lidxb[row, pl.ds(g * 16, 16)] = lv
            return carry
        lax.fori_loop(lo, hi, body, 0)

    def issue_streams(lo, hi, valb, lidxb, spb):
        def body(row, carry):
            pltpu.async_copy(valb.at[row], spb.at[lidxb.at[row]], ssem,
                             add=True)
            return carry
        lax.fori_loop(lo, hi, body, 0)

    def drain_streams(lo, hi, valb, lidxb, spb):
        def body(row, carry):
            pltpu.make_async_copy(valb.at[row], spb.at[lidxb.at[row]],
                                  ssem).wait()
            return carry
        lax.fori_loop(lo, hi, body, 0)

    r0s = [pl.multiple_of(jnp.minimum((bv0[k] >> 10) << 3, _PROWS - _NR), 8)
           for k in range(_NCHUNK)]
    nrs = [((bv1[k] + 127) >> 7) - r0s[k] for k in range(_NCHUNK)]

    start_dense_load(0)
    start_pair_load(0, r0s[0])
    for d in pld_desc[0]:
        d.wait()
    compute_lidx(0, jnp.minimum(nrs[0], _NR), idxs[0], lidxs[0],
                 chunk_lo_of(0) + sl_lo)

    for k in range(_NCHUNK):
        cur = k % 2
        nxt = 1 - cur
        if k + 1 < _NCHUNK:
            if dst_desc[nxt] is not None:
                dst_desc[nxt].wait()
                dst_desc[nxt] = None
            start_dense_load(k + 1)
            start_pair_load(k + 1, r0s[k + 1])

        blk_lo = chunk_lo_of(k) + sl_lo
        nr = nrs[k]
        nr1 = jnp.minimum(nr, _NR)

        dld_desc[cur].wait()
        issue_streams(0, nr1, vals[cur], lidxs[cur], sps[cur])

        if k + 1 < _NCHUNK:
            for d in pld_desc[nxt]:
                d.wait()
            pld_desc[nxt] = None
            compute_lidx(0, jnp.minimum(nrs[k + 1], _NR), idxs[nxt],
                         lidxs[nxt], chunk_lo_of(k + 1) + sl_lo)

        drain_streams(0, nr1, vals[cur], lidxs[cur], sps[cur])

        nbatch = (nr + _NR - 1) // _NR

        def rem_body(j, carry, cur=cur, r0=r0s[k], nr=nr, blk_lo=blk_lo):
            b0 = r0 + j * _NR
            rb = pl.multiple_of(jnp.minimum(b0, _PROWS - _NR), 8)
            ov = b0 - rb
            pltpu.sync_copy(idx2d_hbm.at[pl.ds(rb, _NR)], idxs[cur])
            pltpu.sync_copy(val2d_hbm.at[pl.ds(rb, _NR)], vals[cur])
            hi = ov + jnp.minimum(nr - j * _NR, _NR)
            compute_lidx(ov, hi, idxs[cur], lidxs[cur], blk_lo)
            issue_streams(ov, hi, vals[cur], lidxs[cur], sps[cur])
            drain_streams(ov, hi, vals[cur], lidxs[cur], sps[cur])
            return carry

        lax.fori_loop(1, nbatch, rem_body, 0)

        dst_desc[cur] = pltpu.async_copy(
            sps[cur].at[pl.ds(sl_lo, _BLK)],
            out_hbm.at[pl.ds(blk_lo, _BLK)], dst[cur])

    for d in dst_desc:
        if d is not None:
            d.wait()


def kernel(tensor, values, indices):
    flat = tensor.reshape(-1)
    queries = jnp.arange(_NBLK + 1, dtype=jnp.int32) * _BLK
    bounds = jnp.searchsorted(indices, queries, side="left").astype(jnp.int32)
    w = jnp.arange(_NW)
    s_ = w // _NC
    c_ = w % _NC
    k_ = jnp.arange(_NCHUNK)
    ids = c_[:, None] * (_NBLK // _NC) + k_[None, :] * _NS + s_[:, None]
    table = jnp.concatenate([bounds[ids], bounds[ids + 1]],
                            axis=1).reshape(-1)

    idx2d = indices.reshape(_PROWS, 128)
    val2d = values.reshape(_PROWS, 128)

    out = _sc_scatter_add(flat, val2d, idx2d, table)
    return out.reshape(_SHAPE)
